# baseline (speedup 1.0000x reference)
import functools

import jax
import jax.numpy as jnp
from jax import lax
from jax.experimental import pallas as pl
from jax.experimental.pallas import tpu as pltpu
from jax.experimental.pallas import tpu_sc as plsc

N0 = 10000
N1 = 5000
N2 = 2000
E0 = 320000
E1 = 160000
D = 128
H = 256
O = 64

NC = 2
NS = 16
NW = NC * NS
L = 16

N1P = 5120
N2P = 2048
DEGW = 8
ZR = 16


CHUNK = 64
GS = 4


@functools.lru_cache(maxsize=None)
def _make_segsum(n_tgt_pad: int, iters: int):
    assert iters % GS == 0
    rows_per_sub = n_tgt_pad // NS
    assert rows_per_sub % ZR == 0

    mesh = plsc.VectorSubcoreMesh(core_axis_name="c", subcore_axis_name="s")

    @functools.partial(
        pl.kernel,
        mesh=mesh,
        out_type=[
            jax.ShapeDtypeStruct((NC, n_tgt_pad, D), jnp.float32),
            jax.ShapeDtypeStruct((NC, n_tgt_pad, DEGW), jnp.float32),
        ],
        scratch_types=[
            [pltpu.VMEM((CHUNK,), jnp.int32)] * GS,
            [pltpu.VMEM((CHUNK,), jnp.int32)] * GS,
            [pltpu.VMEM((CHUNK, D), jnp.float32)] * GS,
            pltpu.VMEM((CHUNK, DEGW), jnp.float32),
            pltpu.VMEM((ZR, D), jnp.float32),
            pltpu.VMEM((ZR, DEGW), jnp.float32),
            pltpu.VMEM_SHARED((n_tgt_pad, D), jnp.float32),
            pltpu.VMEM_SHARED((n_tgt_pad, DEGW), jnp.float32),
            pltpu.SemaphoreType.DMA,
        ],
    )
    def k(table, eidx, acc_out, deg_out,
          src_v, dst_v, rows_v, ones_v, zrow_v, zdeg_v, acc_sh, deg_sh, sem):
        cid = lax.axis_index("c")
        sid = lax.axis_index("s")
        wid = sid * NC + cid

        z16 = jnp.zeros((L,), jnp.float32)
        o16 = jnp.ones((L,), jnp.float32)

        def fill_zrow(i, _):
            r = i // (D // L)
            c = (i % (D // L)) * L
            zrow_v[r, pl.ds(c, L)] = z16
            return 0
        lax.fori_loop(0, ZR * (D // L), fill_zrow, 0)

        def fill_zdeg(i, _):
            zdeg_v[i // 2, pl.ds((i % 2) * DEGW, DEGW)] = z16[:DEGW]
            return 0
        lax.fori_loop(0, ZR * 2, fill_zdeg, 0)

        def fill_ones(i, _):
            ones_v[i // 2, pl.ds((i % 2) * DEGW, DEGW)] = o16[:DEGW]
            return 0
        lax.fori_loop(0, CHUNK * 2, fill_ones, 0)

        base_r = sid * rows_per_sub

        def zero_acc(i, _):
            pltpu.sync_copy(zrow_v, acc_sh.at[pl.ds(base_r + i * ZR, ZR)])
            pltpu.sync_copy(zdeg_v, deg_sh.at[pl.ds(base_r + i * ZR, ZR)])
            return 0
        lax.fori_loop(0, rows_per_sub // ZR, zero_acc, 0)

        plsc.subcore_barrier()

        n_groups = iters // GS
        gbase = wid * n_groups * 2 * GS * CHUNK

        def body(g, _):
            off = gbase + g * (2 * GS * CHUNK)
            for j in range(GS):
                pltpu.sync_copy(eidx.at[pl.ds(off + 2 * j * CHUNK, CHUNK)],
                                src_v[j])
                pltpu.sync_copy(
                    eidx.at[pl.ds(off + (2 * j + 1) * CHUNK, CHUNK)],
                    dst_v[j])
            cps = [pltpu.async_copy(table.at[src_v[j]], rows_v[j], sem)
                   for j in range(GS)]
            for cp in cps:
                cp.wait()
            for j in range(GS):
                pltpu.sync_copy(rows_v[j], acc_sh.at[dst_v[j]], add=True)
                pltpu.sync_copy(ones_v, deg_sh.at[dst_v[j]], add=True)
            return 0
        lax.fori_loop(0, n_groups, body, 0)

        plsc.subcore_barrier()

        pltpu.sync_copy(acc_sh.at[pl.ds(base_r, rows_per_sub)],
                        acc_out.at[cid, pl.ds(base_r, rows_per_sub)])
        pltpu.sync_copy(deg_sh.at[pl.ds(base_r, rows_per_sub)],
                        deg_out.at[cid, pl.ds(base_r, rows_per_sub)])

    return k


def _prep_edges(edge_index, iters, pad_dst):
    e = edge_index.shape[1]
    e_pad = NW * iters * CHUNK
    src = edge_index[0].astype(jnp.int32)
    dst = edge_index[1].astype(jnp.int32)
    src = jnp.concatenate([src, jnp.zeros((e_pad - e,), jnp.int32)])
    dst = jnp.concatenate([dst, jnp.full((e_pad - e,), pad_dst, jnp.int32)])
    idx = jnp.stack([src, dst], 0).reshape(2, NW, iters, CHUNK)
    return idx.transpose(1, 2, 0, 3).reshape(-1)


def _mlp1_body(a0, a1, d0, d1, w, b, out):
    deg = d0[:, 0:1] + d1[:, 0:1]
    a = (a0[...] + a1[...]) / jnp.maximum(deg, 1.0)
    y = jnp.dot(a, w[...], preferred_element_type=jnp.float32) + b[...]
    n = jnp.sqrt(jnp.sum(y * y, axis=-1, keepdims=True))
    out[...] = y / jnp.maximum(n, 1e-12)


def _mlp1(acc, deg, W1, b1):
    BR = 640
    grid = N1P // BR
    return pl.pallas_call(
        _mlp1_body,
        grid=(grid,),
        in_specs=[
            pl.BlockSpec((BR, D), lambda i: (i, 0)),
            pl.BlockSpec((BR, D), lambda i: (i, 0)),
            pl.BlockSpec((BR, DEGW), lambda i: (i, 0)),
            pl.BlockSpec((BR, DEGW), lambda i: (i, 0)),
            pl.BlockSpec((D, D), lambda i: (0, 0)),
            pl.BlockSpec((1, D), lambda i: (0, 0)),
        ],
        out_specs=pl.BlockSpec((BR, D), lambda i: (i, 0)),
        out_shape=jax.ShapeDtypeStruct((N1P, D), jnp.float32),
    )(acc[0], acc[1], deg[0], deg[1], W1, b1)


def _mlp2_body(a0, a1, d0, d1, wa, ba, wb, bb, out):
    deg = d0[:, 0:1] + d1[:, 0:1]
    a = (a0[...] + a1[...]) / jnp.maximum(deg, 1.0)
    y = jnp.dot(a, wa[...], preferred_element_type=jnp.float32) + ba[...]
    y = jnp.maximum(y, 0.0)
    z = jnp.dot(y, wb[...], preferred_element_type=jnp.float32) + bb[...]
    n = jnp.sqrt(jnp.sum(z * z, axis=-1, keepdims=True))
    out[...] = z / jnp.maximum(n, 1e-12)


def _mlp2(acc, deg, W2a, b2a, W2b, b2b):
    BR = 512
    grid = N2P // BR
    return pl.pallas_call(
        _mlp2_body,
        grid=(grid,),
        in_specs=[
            pl.BlockSpec((BR, D), lambda i: (i, 0)),
            pl.BlockSpec((BR, D), lambda i: (i, 0)),
            pl.BlockSpec((BR, DEGW), lambda i: (i, 0)),
            pl.BlockSpec((BR, DEGW), lambda i: (i, 0)),
            pl.BlockSpec((D, H), lambda i: (0, 0)),
            pl.BlockSpec((1, H), lambda i: (0, 0)),
            pl.BlockSpec((H, O), lambda i: (0, 0)),
            pl.BlockSpec((1, O), lambda i: (0, 0)),
        ],
        out_specs=pl.BlockSpec((BR, O), lambda i: (i, 0)),
        out_shape=jax.ShapeDtypeStruct((N2P, O), jnp.float32),
    )(acc[0], acc[1], deg[0], deg[1], W2a, b2a, W2b, b2b)


ITERS0 = 160
ITERS1 = 80


def kernel(x, edge_index0, edge_index1, W1, b1, W2a, b2a, W2b, b2b):
    eidx0 = _prep_edges(edge_index0, ITERS0, N1P - 1)
    eidx1 = _prep_edges(edge_index1, ITERS1, N2P - 1)

    acc0, deg0 = _make_segsum(N1P, ITERS0)(x, eidx0)
    h = _mlp1(acc0, deg0, W1, b1.reshape(1, D))
    acc1, deg1 = _make_segsum(N2P, ITERS1)(h, eidx1)
    out = _mlp2(acc1, deg1, W2a, b2a.reshape(1, H), W2b, b2b.reshape(1, O))
    return out[:N2]


# speedup vs baseline: 2.2172x; 2.2172x over previous
import functools

import jax
import jax.numpy as jnp
from jax import lax
from jax.experimental import pallas as pl
from jax.experimental.pallas import tpu as pltpu
from jax.experimental.pallas import tpu_sc as plsc

N0 = 10000
N1 = 5000
N2 = 2000
E0 = 320000
E1 = 160000
D = 128
H = 256
O = 64

NC = 2
NS = 16
NW = NC * NS
L = 16

N1P = 5120
N2P = 2048
DEGW = 8
ZR = 16

CHUNK = 112
GS = 5
ITERS0 = 90
ITERS1 = 45
DG = 9


def _mesh():
    return plsc.VectorSubcoreMesh(core_axis_name="c", subcore_axis_name="s")


def _fill_rows(ref, n_rows, width, value):
    v16 = jnp.full((L,), value, jnp.float32)
    if width >= L:
        per_row = width // L

        def body(i, _):
            ref[i // per_row, pl.ds((i % per_row) * L, L)] = v16
            return 0
        lax.fori_loop(0, n_rows * per_row, body, 0)
    else:
        rows_per_store = L // width

        def body(i, _):
            ref[i // rows_per_store,
                pl.ds((i % rows_per_store) * width, width)] = v16[:width]
            return 0
        lax.fori_loop(0, n_rows * rows_per_store, body, 0)


@functools.lru_cache(maxsize=None)
def _make_segsum(n_tgt_pad: int, iters: int):
    assert iters % GS == 0
    rows_per_sub = n_tgt_pad // NS
    assert rows_per_sub % ZR == 0

    @functools.partial(
        pl.kernel,
        mesh=_mesh(),
        out_type=jax.ShapeDtypeStruct((NC, n_tgt_pad, D), jnp.float32),
        scratch_types=[
            [pltpu.VMEM((CHUNK,), jnp.int32)] * GS,
            [pltpu.VMEM((CHUNK,), jnp.int32)] * GS,
            [pltpu.VMEM((CHUNK, D), jnp.float32)] * GS,
            pltpu.VMEM((ZR, D), jnp.float32),
            pltpu.VMEM_SHARED((n_tgt_pad, D), jnp.float32),
            pltpu.SemaphoreType.DMA,
            pltpu.SemaphoreType.DMA,
            pltpu.SemaphoreType.DMA,
        ],
    )
    def k(table, eidx, acc_out,
          src_v, dst_v, rows_v, zrow_v, acc_sh, sem_i, sem_g, sem_s):
        cid = lax.axis_index("c")
        sid = lax.axis_index("s")
        wid = sid * NC + cid

        _fill_rows(zrow_v, ZR, D, 0.0)

        base_r = sid * rows_per_sub

        def zero_acc(i, _):
            pltpu.sync_copy(zrow_v, acc_sh.at[pl.ds(base_r + i * ZR, ZR)])
            return 0
        lax.fori_loop(0, rows_per_sub // ZR, zero_acc, 0)

        plsc.subcore_barrier()

        n_groups = iters // GS
        wbase = wid * iters * 2 * CHUNK

        def body(g, _):
            t0 = g * GS
            ics = []
            for j in range(GS):
                cb = wbase + (t0 + j) * 2 * CHUNK
                ics.append(pltpu.async_copy(
                    eidx.at[pl.ds(cb, CHUNK)], src_v[j], sem_i))
                ics.append(pltpu.async_copy(
                    eidx.at[pl.ds(cb + CHUNK, CHUNK)], dst_v[j], sem_i))
            for cp in ics:
                cp.wait()
            cps = [pltpu.async_copy(table.at[src_v[j]], rows_v[j], sem_g)
                   for j in range(GS)]
            for cp in cps:
                cp.wait()
            scs = [pltpu.async_copy(rows_v[j], acc_sh.at[dst_v[j]], sem_s,
                                    add=True)
                   for j in range(GS)]
            for cp in scs:
                cp.wait()
            return 0
        lax.fori_loop(0, n_groups, body, 0)

        plsc.subcore_barrier()

        pltpu.sync_copy(acc_sh.at[pl.ds(base_r, rows_per_sub)],
                        acc_out.at[cid, pl.ds(base_r, rows_per_sub)])

    return k


@functools.lru_cache(maxsize=None)
def _make_degrees():
    r0 = N1P // NS
    r1 = N2P // NS

    @functools.partial(
        pl.kernel,
        mesh=_mesh(),
        out_type=[
            jax.ShapeDtypeStruct((NC, N1P, DEGW), jnp.float32),
            jax.ShapeDtypeStruct((NC, N2P, DEGW), jnp.float32),
        ],
        scratch_types=[
            [pltpu.VMEM((CHUNK,), jnp.int32)] * DG,
            pltpu.VMEM((CHUNK, DEGW), jnp.float32),
            pltpu.VMEM((ZR, DEGW), jnp.float32),
            pltpu.VMEM_SHARED((N1P, DEGW), jnp.float32),
            pltpu.VMEM_SHARED((N2P, DEGW), jnp.float32),
            pltpu.SemaphoreType.DMA,
            pltpu.SemaphoreType.DMA,
        ],
    )
    def k(edst0, edst1, deg0_out, deg1_out,
          dst_v, ones_v, zdeg_v, deg0_sh, deg1_sh, sem_i, sem_s):
        cid = lax.axis_index("c")
        sid = lax.axis_index("s")
        wid = sid * NC + cid

        _fill_rows(ones_v, CHUNK, DEGW, 1.0)
        _fill_rows(zdeg_v, ZR, DEGW, 0.0)

        def zero0(i, _):
            pltpu.sync_copy(zdeg_v, deg0_sh.at[pl.ds(sid * r0 + i * ZR, ZR)])
            return 0
        lax.fori_loop(0, r0 // ZR, zero0, 0)

        def zero1(i, _):
            pltpu.sync_copy(zdeg_v, deg1_sh.at[pl.ds(sid * r1 + i * ZR, ZR)])
            return 0
        lax.fori_loop(0, r1 // ZR, zero1, 0)

        plsc.subcore_barrier()

        def layer(edst, deg_sh, iters):
            wbase = wid * iters * CHUNK
            n_groups = iters // DG

            def body(g, _):
                t0 = g * DG
                ics = [pltpu.async_copy(
                    edst.at[pl.ds(wbase + (t0 + j) * CHUNK, CHUNK)],
                    dst_v[j], sem_i)
                    for j in range(DG)]
                for cp in ics:
                    cp.wait()
                scs = [pltpu.async_copy(ones_v, deg_sh.at[dst_v[j]], sem_s,
                                        add=True)
                       for j in range(DG)]
                for cp in scs:
                    cp.wait()
                return 0
            lax.fori_loop(0, n_groups, body, 0)

        layer(edst0, deg0_sh, ITERS0)
        layer(edst1, deg1_sh, ITERS1)

        plsc.subcore_barrier()

        pltpu.sync_copy(deg0_sh.at[pl.ds(sid * r0, r0)],
                        deg0_out.at[cid, pl.ds(sid * r0, r0)])
        pltpu.sync_copy(deg1_sh.at[pl.ds(sid * r1, r1)],
                        deg1_out.at[cid, pl.ds(sid * r1, r1)])

    return k


def _prep_edges(edge_index, iters, pad_dst):
    e = edge_index.shape[1]
    e_pad = NW * iters * CHUNK
    src = edge_index[0].astype(jnp.int32)
    dst = edge_index[1].astype(jnp.int32)
    src = jnp.concatenate([src, jnp.zeros((e_pad - e,), jnp.int32)])
    dst = jnp.concatenate([dst, jnp.full((e_pad - e,), pad_dst, jnp.int32)])
    idx = jnp.stack([src, dst], 0).reshape(2, NW, iters, CHUNK)
    both = idx.transpose(1, 2, 0, 3).reshape(-1)
    dst_only = idx[1].reshape(-1)
    return both, dst_only


def _mlp1_body(a0, a1, d0, d1, w, b, out):
    deg = d0[:, 0:1] + d1[:, 0:1]
    a = (a0[...] + a1[...]) / jnp.maximum(deg, 1.0)
    y = jnp.dot(a, w[...], preferred_element_type=jnp.float32) + b[...]
    n = jnp.sqrt(jnp.sum(y * y, axis=-1, keepdims=True))
    out[...] = y / jnp.maximum(n, 1e-12)


def _mlp1(acc, deg, W1, b1):
    BR = 640
    grid = N1P // BR
    return pl.pallas_call(
        _mlp1_body,
        grid=(grid,),
        in_specs=[
            pl.BlockSpec((BR, D), lambda i: (i, 0)),
            pl.BlockSpec((BR, D), lambda i: (i, 0)),
            pl.BlockSpec((BR, DEGW), lambda i: (i, 0)),
            pl.BlockSpec((BR, DEGW), lambda i: (i, 0)),
            pl.BlockSpec((D, D), lambda i: (0, 0)),
            pl.BlockSpec((1, D), lambda i: (0, 0)),
        ],
        out_specs=pl.BlockSpec((BR, D), lambda i: (i, 0)),
        out_shape=jax.ShapeDtypeStruct((N1P, D), jnp.float32),
    )(acc[0], acc[1], deg[0], deg[1], W1, b1)


def _mlp2_body(a0, a1, d0, d1, wa, ba, wb, bb, out):
    deg = d0[:, 0:1] + d1[:, 0:1]
    a = (a0[...] + a1[...]) / jnp.maximum(deg, 1.0)
    y = jnp.dot(a, wa[...], preferred_element_type=jnp.float32) + ba[...]
    y = jnp.maximum(y, 0.0)
    z = jnp.dot(y, wb[...], preferred_element_type=jnp.float32) + bb[...]
    n = jnp.sqrt(jnp.sum(z * z, axis=-1, keepdims=True))
    out[...] = z / jnp.maximum(n, 1e-12)


def _mlp2(acc, deg, W2a, b2a, W2b, b2b):
    BR = 512
    grid = N2P // BR
    return pl.pallas_call(
        _mlp2_body,
        grid=(grid,),
        in_specs=[
            pl.BlockSpec((BR, D), lambda i: (i, 0)),
            pl.BlockSpec((BR, D), lambda i: (i, 0)),
            pl.BlockSpec((BR, DEGW), lambda i: (i, 0)),
            pl.BlockSpec((BR, DEGW), lambda i: (i, 0)),
            pl.BlockSpec((D, H), lambda i: (0, 0)),
            pl.BlockSpec((1, H), lambda i: (0, 0)),
            pl.BlockSpec((H, O), lambda i: (0, 0)),
            pl.BlockSpec((1, O), lambda i: (0, 0)),
        ],
        out_specs=pl.BlockSpec((BR, O), lambda i: (i, 0)),
        out_shape=jax.ShapeDtypeStruct((N2P, O), jnp.float32),
    )(acc[0], acc[1], deg[0], deg[1], W2a, b2a, W2b, b2b)


def kernel(x, edge_index0, edge_index1, W1, b1, W2a, b2a, W2b, b2b):
    eidx0, edst0 = _prep_edges(edge_index0, ITERS0, N1P - 1)
    eidx1, edst1 = _prep_edges(edge_index1, ITERS1, N2P - 1)

    deg0, deg1 = _make_degrees()(edst0, edst1)
    acc0 = _make_segsum(N1P, ITERS0)(x, eidx0)
    h = _mlp1(acc0, deg0, W1, b1.reshape(1, D))
    acc1 = _make_segsum(N2P, ITERS1)(h, eidx1)
    out = _mlp2(acc1, deg1, W2a, b2a.reshape(1, H), W2b, b2b.reshape(1, O))
    return out[:N2]
